# baseline (speedup 1.0000x reference)
import functools

import jax
import jax.numpy as jnp
from jax import lax
from jax.experimental import pallas as pl
from jax.experimental.pallas import tpu as pltpu
from jax.experimental.pallas import tpu_sc as plsc

N = 10000
E = 160000
NC, NS, L = 2, 16, 16
CHUNK = 128
EPAD = NC * NS * CHUNK * 40
NCHUNKS = EPAD // CHUNK
CPW_HALF = NCHUNKS // (NC * NS)
CPW_FULL = NCHUNKS // NS
KB = 8
NIB = CPW_FULL // KB
NPAIR = NIB // 2
TRASH = N
ROWS_A = 10240
D1 = 16
D2 = 128
BLK = 1000

_mesh = plsc.VectorSubcoreMesh(
    core_axis_name="c", subcore_axis_name="s", num_cores=NC, num_subcores=NS)

_f32 = jnp.float32
_HI = lax.Precision.HIGHEST
_sc_params = pltpu.CompilerParams(use_tc_tiling_on_sc=False)


def _fill_rows(buf, width, value):
    @pl.loop(0, CHUNK)
    def _(i):
        @pl.loop(0, width // L)
        def _(j):
            buf[i, pl.ds(j * L, L)] = jnp.full((L,), value, _f32)


def _zero_accum(zero_v, accum, s):
    @pl.loop(0, 5)
    def _(k):
        pltpu.sync_copy(zero_v, accum.at[pl.ds(s * 640 + k * CHUNK, CHUNK)])


def _sc_deg(dst2d):
    @functools.partial(
        pl.kernel,
        out_type=jax.ShapeDtypeStruct((NC, ROWS_A, D1), _f32),
        mesh=_mesh,
        scratch_types=[
            pltpu.VMEM((CPW_HALF, CHUNK), jnp.int32),
            pltpu.VMEM((CHUNK, D1), _f32),
            pltpu.VMEM_SHARED((ROWS_A, D1), _f32),
        ],
        compiler_params=_sc_params,
    )
    def k(dst_hbm, out_hbm, idx_v, ones_v, accum):
        c = lax.axis_index("c")
        s = lax.axis_index("s")
        wc = c * NS + s
        _fill_rows(ones_v, D1, 0.0)
        _zero_accum(ones_v, accum, s)
        _fill_rows(ones_v, D1, 1.0)
        pltpu.sync_copy(dst_hbm.at[pl.ds(wc * CPW_HALF, CPW_HALF)], idx_v)
        plsc.subcore_barrier()

        @pl.loop(0, CPW_HALF)
        def _(j):
            pltpu.sync_copy(ones_v, accum.at[idx_v.at[j]], add=True)

        plsc.subcore_barrier()
        pltpu.sync_copy(accum.at[pl.ds(s * 640, 640)],
                        out_hbm.at[c].at[pl.ds(s * 640, 640)])

    return k(dst2d)


def _sc_narrow(xs16, src2d, dst2d):
    @functools.partial(
        pl.kernel,
        out_type=jax.ShapeDtypeStruct((NC, ROWS_A, D1), _f32),
        mesh=_mesh,
        scratch_types=[
            pltpu.VMEM((CPW_HALF, CHUNK), jnp.int32),
            pltpu.VMEM((CPW_HALF, CHUNK), jnp.int32),
            pltpu.VMEM((CHUNK, D1), _f32),
            pltpu.VMEM_SHARED((ROWS_A, D1), _f32),
        ],
        compiler_params=_sc_params,
    )
    def k(xs_hbm, src_hbm, dst_hbm, out_hbm, isrc_v, idst_v, rows_v, accum):
        c = lax.axis_index("c")
        s = lax.axis_index("s")
        wc = c * NS + s
        _fill_rows(rows_v, D1, 0.0)
        _zero_accum(rows_v, accum, s)
        pltpu.sync_copy(src_hbm.at[pl.ds(wc * CPW_HALF, CPW_HALF)], isrc_v)
        pltpu.sync_copy(dst_hbm.at[pl.ds(wc * CPW_HALF, CPW_HALF)], idst_v)
        plsc.subcore_barrier()

        @pl.loop(0, CPW_HALF)
        def _(j):
            pltpu.sync_copy(xs_hbm.at[isrc_v.at[j]], rows_v)
            pltpu.sync_copy(rows_v, accum.at[idst_v.at[j]], add=True)

        plsc.subcore_barrier()
        pltpu.sync_copy(accum.at[pl.ds(s * 640, 640)],
                        out_hbm.at[c].at[pl.ds(s * 640, 640)])

    return k(xs16, src2d, dst2d)


def _sc_wide(hs, src2d, dst2d):
    @functools.partial(
        pl.kernel,
        out_type=jax.ShapeDtypeStruct((NC, ROWS_A, D2), _f32),
        mesh=_mesh,
        scratch_types=[
            pltpu.VMEM((2, KB, CHUNK), jnp.int32),
            pltpu.VMEM((2, KB, CHUNK), jnp.int32),
            pltpu.VMEM((CHUNK, D2), _f32),
            pltpu.VMEM((CHUNK, D2), _f32),
            pltpu.VMEM_SHARED((ROWS_A, D2), _f32),
            pltpu.SemaphoreType.DMA,
            pltpu.SemaphoreType.DMA,
            pltpu.SemaphoreType.DMA,
            pltpu.SemaphoreType.DMA,
            pltpu.SemaphoreType.DMA,
        ],
    )
    def k(hs_hbm, src_hbm, dst_hbm, out_hbm, isrc2, idst2,
          rows_a, rows_b, accum, gsem_a, gsem_b, ssem_a, ssem_b, isem):
        c = lax.axis_index("c")
        s = lax.axis_index("s")
        bufs = (rows_a, rows_b)
        gsems = (gsem_a, gsem_b)
        ssems = (ssem_a, ssem_b)

        def g_start(irow, b):
            pltpu.async_copy(hs_hbm.at[c].at[irow], bufs[b], gsems[b])

        def g_wait(b):
            pltpu.make_async_copy(
                hs_hbm.at[c].at[isrc2.at[0].at[0]], bufs[b], gsems[b]).wait()

        def s_start(b, ib, r):
            pltpu.async_copy(bufs[b], accum.at[idst2.at[ib].at[r]],
                             ssems[b], add=True)

        def s_wait(b):
            pltpu.make_async_copy(bufs[b], accum.at[idst2.at[0].at[0]],
                                  ssems[b]).wait()

        def i_start(blk, q):
            base = s * CPW_FULL + blk * KB
            pltpu.async_copy(src_hbm.at[pl.ds(base, KB)], isrc2.at[q], isem)
            pltpu.async_copy(dst_hbm.at[pl.ds(base, KB)], idst2.at[q], isem)

        def i_wait():
            pltpu.make_async_copy(
                src_hbm.at[pl.ds(s * CPW_FULL, KB)], isrc2.at[0], isem).wait()
            pltpu.make_async_copy(
                dst_hbm.at[pl.ds(s * CPW_FULL, KB)], idst2.at[0], isem).wait()

        def pair_body(pair, first, last):
            for cc in range(2 * KB):
                b = cc % 2
                ib = cc // KB
                g_wait(b)
                s_start(b, ib, cc % KB)
                if not (first and cc == 0):
                    s_wait(1 - b)
                if cc == 0:
                    i_start(2 * pair + 1, 1)
                if cc == KB and not last:
                    i_start(2 * pair + 2, 0)
                if cc == 2 * KB - 1:
                    if last:
                        s_wait(b)
                    else:
                        i_wait()
                        g_start(isrc2.at[0].at[0], 1 - b)
                else:
                    if cc == KB - 1:
                        i_wait()
                    nxt = cc + 1
                    g_start(isrc2.at[nxt // KB].at[nxt % KB], 1 - b)

        _fill_rows(rows_a, D2, 0.0)
        _zero_accum(rows_a, accum, s)
        plsc.subcore_barrier()

        pltpu.sync_copy(src_hbm.at[pl.ds(s * CPW_FULL, KB)], isrc2.at[0])
        pltpu.sync_copy(dst_hbm.at[pl.ds(s * CPW_FULL, KB)], idst2.at[0])
        g_start(isrc2.at[0].at[0], 0)

        pair_body(0, True, False)

        @pl.loop(1, NPAIR - 1)
        def _(p):
            pair_body(p, False, False)

        pair_body(NPAIR - 1, False, True)

        plsc.subcore_barrier()
        pltpu.sync_copy(accum.at[pl.ds(s * 640, 640)],
                        out_hbm.at[c].at[pl.ds(s * 640, 640)])

    return k(hs, src2d, dst2d)


def _tc1(degAB, x):
    def body(dab_ref, x_ref, deg1_ref, xs16_ref):
        d = dab_ref[0] + dab_ref[1] + 1.0
        deg1_ref[...] = d
        xs16_ref[...] = x_ref[...] * lax.rsqrt(d)

    return pl.pallas_call(
        body,
        grid=(N // BLK,),
        in_specs=[
            pl.BlockSpec((NC, BLK, D1), lambda i: (0, i, 0)),
            pl.BlockSpec((BLK, 1), lambda i: (i, 0)),
        ],
        out_specs=[
            pl.BlockSpec((BLK, D1), lambda i: (i, 0)),
            pl.BlockSpec((BLK, D1), lambda i: (i, 0)),
        ],
        out_shape=[
            jax.ShapeDtypeStruct((N, D1), _f32),
            jax.ShapeDtypeStruct((N, D1), _f32),
        ],
    )(degAB, x)


def _tc2(s1AB, deg1, x, W1, b1):
    def body(s1_ref, deg1_ref, x_ref, w1_ref, b1_ref, out_ref):
        dcol = lax.rsqrt(deg1_ref[...][:, :1])
        xs = x_ref[...] * dcol
        s1 = s1_ref[0][:, :1] + s1_ref[1][:, :1]
        m1 = dcol * (s1 + xs)
        h1 = jnp.maximum(m1 * w1_ref[...][None, :] + b1_ref[...][None, :], 0.0)
        hs1 = h1 * dcol
        out_ref[0] = hs1[:, :D2]
        out_ref[1] = hs1[:, D2:]

    return pl.pallas_call(
        body,
        grid=(N // BLK,),
        in_specs=[
            pl.BlockSpec((NC, BLK, D1), lambda i: (0, i, 0)),
            pl.BlockSpec((BLK, D1), lambda i: (i, 0)),
            pl.BlockSpec((BLK, 1), lambda i: (i, 0)),
            pl.BlockSpec((256,), lambda i: (0,)),
            pl.BlockSpec((256,), lambda i: (0,)),
        ],
        out_specs=pl.BlockSpec((NC, BLK, D2), lambda i: (0, i, 0)),
        out_shape=jax.ShapeDtypeStruct((NC, N, D2), _f32),
    )(s1AB, deg1, x, W1, b1)


def _tc3(m2, hs1, deg1, W2, b2):
    def body(m_ref, hs_ref, deg1_ref, w_ref, b_ref, out_ref):
        dcol = lax.rsqrt(deg1_ref[...][:, :1])
        p = dcol * jnp.concatenate(
            [m_ref[0] + hs_ref[0], m_ref[1] + hs_ref[1]], axis=1)
        h = jnp.maximum(
            jnp.dot(p, w_ref[...], preferred_element_type=_f32, precision=_HI)
            + b_ref[...][None, :], 0.0)
        hs = h * dcol
        out_ref[0] = hs[:, :D2]
        out_ref[1] = hs[:, D2:]

    return pl.pallas_call(
        body,
        grid=(N // BLK,),
        in_specs=[
            pl.BlockSpec((NC, BLK, D2), lambda i: (0, i, 0)),
            pl.BlockSpec((NC, BLK, D2), lambda i: (0, i, 0)),
            pl.BlockSpec((BLK, D1), lambda i: (i, 0)),
            pl.BlockSpec((256, 256), lambda i: (0, 0)),
            pl.BlockSpec((256,), lambda i: (0,)),
        ],
        out_specs=pl.BlockSpec((NC, BLK, D2), lambda i: (0, i, 0)),
        out_shape=jax.ShapeDtypeStruct((NC, N, D2), _f32),
    )(m2, hs1, deg1, W2, b2)


def _ln_silu(z, g, b, eps=1e-5):
    m = jnp.mean(z, axis=-1, keepdims=True)
    v = jnp.mean((z - m) ** 2, axis=-1, keepdims=True)
    z = (z - m) * lax.rsqrt(v + eps) * g[None, :] + b[None, :]
    return z * jax.nn.sigmoid(z)


def _tc4(m3, hs2, deg1, vecp, Wvp, bv, W3, b3, WaggT, bagg,
         g1, be1, Wc1, bc1, g2, be2, Wc2, bc2, g3, be3):
    def body(m_ref, hs_ref, deg1_ref, vec_ref, wv_ref, bv_ref, w3_ref, b3_ref,
             wa_ref, ba_ref, g1_ref, be1_ref, wc1_ref, bc1_ref,
             g2_ref, be2_ref, wc2_ref, bc2_ref, g3_ref, be3_ref, out_ref):
        dcol = lax.rsqrt(deg1_ref[...][:, :1])
        p = dcol * jnp.concatenate(
            [m_ref[0] + hs_ref[0], m_ref[1] + hs_ref[1]], axis=1)
        h3 = jnp.maximum(
            jnp.dot(p, w3_ref[...], preferred_element_type=_f32, precision=_HI)
            + b3_ref[...][None, :], 0.0)
        agg = jnp.dot(h3, wa_ref[...], preferred_element_type=_f32,
                      precision=_HI) + ba_ref[...][None, :]
        hv = jnp.dot(vec_ref[...], wv_ref[...], preferred_element_type=_f32,
                     precision=_HI) + bv_ref[...][None, :]
        z = jnp.concatenate([agg, hv], axis=1)
        z = _ln_silu(z, g1_ref[...], be1_ref[...])
        z = jnp.dot(z, wc1_ref[...], preferred_element_type=_f32,
                    precision=_HI) + bc1_ref[...][None, :]
        z = _ln_silu(z, g2_ref[...], be2_ref[...])
        z = jnp.dot(z, wc2_ref[...], preferred_element_type=_f32,
                    precision=_HI) + bc2_ref[...][None, :]
        out_ref[...] = _ln_silu(z, g3_ref[...], be3_ref[...])

    full = lambda shape: pl.BlockSpec(shape, lambda i: tuple(0 for _ in shape))
    return pl.pallas_call(
        body,
        grid=(N // BLK,),
        in_specs=[
            pl.BlockSpec((NC, BLK, D2), lambda i: (0, i, 0)),
            pl.BlockSpec((NC, BLK, D2), lambda i: (0, i, 0)),
            pl.BlockSpec((BLK, D1), lambda i: (i, 0)),
            pl.BlockSpec((BLK, 8), lambda i: (i, 0)),
            full((8, 256)), full((256,)),
            full((256, 256)), full((256,)),
            full((256, 32)), full((32,)),
            full((288,)), full((288,)),
            full((288, 512)), full((512,)),
            full((512,)), full((512,)),
            full((512, 512)), full((512,)),
            full((512,)), full((512,)),
        ],
        out_specs=pl.BlockSpec((BLK, 512), lambda i: (i, 0)),
        out_shape=jax.ShapeDtypeStruct((N, 512), _f32),
    )(m3, hs2, deg1, vecp, Wvp, bv, W3, b3, WaggT, bagg,
      g1, be1, Wc1, bc1, g2, be2, Wc2, bc2, g3, be3)


def kernel(x, edge_index, batch_index, vector, W1, b1, W2, b2, W3, b3,
           Wagg, bagg, Wv, bv, g1, be1, Wc1, bc1, g2, be2, Wc2, bc2, g3, be3):
    del batch_index
    src = edge_index[0]
    dst = edge_index[1]
    pad = EPAD - E
    src2d = jnp.concatenate(
        [src, jnp.zeros((pad,), jnp.int32)]).reshape(NCHUNKS, CHUNK)
    dst2d = jnp.concatenate(
        [dst, jnp.full((pad,), TRASH, jnp.int32)]).reshape(NCHUNKS, CHUNK)

    degAB = _sc_deg(dst2d)
    deg1, xs16 = _tc1(degAB, x)
    s1AB = _sc_narrow(xs16, src2d, dst2d)
    hs1 = _tc2(s1AB, deg1, x, W1[0], b1)
    m2 = _sc_wide(hs1, src2d, dst2d)
    hs2 = _tc3(m2, hs1, deg1, W2, b2)
    m3 = _sc_wide(hs2, src2d, dst2d)
    vecp = jnp.pad(vector, ((0, 0), (0, 2)))
    Wvp = jnp.pad(Wv, ((0, 2), (0, 0)))
    return _tc4(m3, hs2, deg1, vecp, Wvp, bv, W3, b3, Wagg[:256], bagg,
                g1, be1, Wc1, bc1, g2, be2, Wc2, bc2, g3, be3)


# speedup vs baseline: 1.1438x; 1.1438x over previous
import functools

import jax
import jax.numpy as jnp
from jax import lax
from jax.experimental import pallas as pl
from jax.experimental.pallas import tpu as pltpu
from jax.experimental.pallas import tpu_sc as plsc

N = 10000
E = 160000
NC, NS, L = 2, 16, 16
CHUNK = 128
EPAD = NC * NS * CHUNK * 40
NCHUNKS = EPAD // CHUNK
CPW_HALF = NCHUNKS // (NC * NS)
CPW_FULL = NCHUNKS // NS
KB = 16
NB = CPW_FULL // KB
TRASH = N
ROWS_A = 10240
D1 = 16
D2 = 128
BLK = 1000

_mesh = plsc.VectorSubcoreMesh(
    core_axis_name="c", subcore_axis_name="s", num_cores=NC, num_subcores=NS)

_f32 = jnp.float32
_HI = lax.Precision.DEFAULT
_sc_params = pltpu.CompilerParams(use_tc_tiling_on_sc=False)


def _fill_rows(buf, width, value):
    @pl.loop(0, CHUNK)
    def _(i):
        @pl.loop(0, width // L)
        def _(j):
            buf[i, pl.ds(j * L, L)] = jnp.full((L,), value, _f32)


def _zero_accum(zero_v, accum, s):
    @pl.loop(0, 5)
    def _(k):
        pltpu.sync_copy(zero_v, accum.at[pl.ds(s * 640 + k * CHUNK, CHUNK)])


def _sc_deg(dst2d):
    @functools.partial(
        pl.kernel,
        out_type=jax.ShapeDtypeStruct((NC, ROWS_A, D1), _f32),
        mesh=_mesh,
        scratch_types=[
            pltpu.VMEM((CPW_HALF, CHUNK), jnp.int32),
            pltpu.VMEM((CHUNK, D1), _f32),
            pltpu.VMEM_SHARED((ROWS_A, D1), _f32),
        ],
        compiler_params=_sc_params,
    )
    def k(dst_hbm, out_hbm, idx_v, ones_v, accum):
        c = lax.axis_index("c")
        s = lax.axis_index("s")
        wc = c * NS + s
        _fill_rows(ones_v, D1, 0.0)
        _zero_accum(ones_v, accum, s)
        _fill_rows(ones_v, D1, 1.0)
        pltpu.sync_copy(dst_hbm.at[pl.ds(wc * CPW_HALF, CPW_HALF)], idx_v)
        plsc.subcore_barrier()

        @pl.loop(0, CPW_HALF)
        def _(j):
            pltpu.sync_copy(ones_v, accum.at[idx_v.at[j]], add=True)

        plsc.subcore_barrier()
        pltpu.sync_copy(accum.at[pl.ds(s * 640, 640)],
                        out_hbm.at[c].at[pl.ds(s * 640, 640)])

    return k(dst2d)


def _sc_narrow(xs16, src2d, dst2d):
    @functools.partial(
        pl.kernel,
        out_type=jax.ShapeDtypeStruct((NC, ROWS_A, D1), _f32),
        mesh=_mesh,
        scratch_types=[
            pltpu.VMEM((CPW_HALF, CHUNK), jnp.int32),
            pltpu.VMEM((CPW_HALF, CHUNK), jnp.int32),
            pltpu.VMEM((CHUNK, D1), _f32),
            pltpu.VMEM_SHARED((ROWS_A, D1), _f32),
        ],
        compiler_params=_sc_params,
    )
    def k(xs_hbm, src_hbm, dst_hbm, out_hbm, isrc_v, idst_v, rows_v, accum):
        c = lax.axis_index("c")
        s = lax.axis_index("s")
        wc = c * NS + s
        _fill_rows(rows_v, D1, 0.0)
        _zero_accum(rows_v, accum, s)
        pltpu.sync_copy(src_hbm.at[pl.ds(wc * CPW_HALF, CPW_HALF)], isrc_v)
        pltpu.sync_copy(dst_hbm.at[pl.ds(wc * CPW_HALF, CPW_HALF)], idst_v)
        plsc.subcore_barrier()

        @pl.loop(0, CPW_HALF)
        def _(j):
            pltpu.sync_copy(xs_hbm.at[isrc_v.at[j]], rows_v)
            pltpu.sync_copy(rows_v, accum.at[idst_v.at[j]], add=True)

        plsc.subcore_barrier()
        pltpu.sync_copy(accum.at[pl.ds(s * 640, 640)],
                        out_hbm.at[c].at[pl.ds(s * 640, 640)])

    return k(xs16, src2d, dst2d)


def _sc_wide(hs, src2d, dst2d):
    @functools.partial(
        pl.kernel,
        out_type=jax.ShapeDtypeStruct((NC, ROWS_A, D2), _f32),
        mesh=_mesh,
        scratch_types=[
            pltpu.VMEM((KB, CHUNK), jnp.int32),
            pltpu.VMEM((KB, CHUNK), jnp.int32),
            pltpu.VMEM((CHUNK, D2), _f32),
            pltpu.VMEM((CHUNK, D2), _f32),
            pltpu.VMEM_SHARED((ROWS_A, D2), _f32),
            pltpu.SemaphoreType.DMA,
            pltpu.SemaphoreType.DMA,
        ],
    )
    def k(hs_hbm, src_hbm, dst_hbm, out_hbm, isrc_v, idst_v,
          rows_a, rows_b, accum, sem_a, sem_b):
        c = lax.axis_index("c")
        s = lax.axis_index("s")
        bufs = (rows_a, rows_b)
        sems = (sem_a, sem_b)
        _fill_rows(rows_a, D2, 0.0)
        _zero_accum(rows_a, accum, s)
        plsc.subcore_barrier()

        @pl.loop(0, NB)
        def _(blk):
            base = s * CPW_FULL + blk * KB
            pltpu.sync_copy(src_hbm.at[pl.ds(base, KB)], isrc_v)
            pltpu.sync_copy(dst_hbm.at[pl.ds(base, KB)], idst_v)
            pltpu.async_copy(hs_hbm.at[c].at[isrc_v.at[0]], bufs[0], sems[0])
            for i in range(KB):
                b = i % 2
                if i + 1 < KB:
                    pltpu.async_copy(hs_hbm.at[c].at[isrc_v.at[i + 1]],
                                     bufs[1 - b], sems[1 - b])
                pltpu.make_async_copy(
                    hs_hbm.at[c].at[isrc_v.at[i]], bufs[b], sems[b]).wait()
                pltpu.sync_copy(bufs[b], accum.at[idst_v.at[i]], add=True)

        plsc.subcore_barrier()
        pltpu.sync_copy(accum.at[pl.ds(s * 640, 640)],
                        out_hbm.at[c].at[pl.ds(s * 640, 640)])

    return k(hs, src2d, dst2d)


def _tc1(degAB, x):
    def body(dab_ref, x_ref, deg1_ref, xs16_ref):
        d = dab_ref[0] + dab_ref[1] + 1.0
        deg1_ref[...] = d
        xs16_ref[...] = x_ref[...] * lax.rsqrt(d)

    return pl.pallas_call(
        body,
        grid=(N // BLK,),
        in_specs=[
            pl.BlockSpec((NC, BLK, D1), lambda i: (0, i, 0)),
            pl.BlockSpec((BLK, 1), lambda i: (i, 0)),
        ],
        out_specs=[
            pl.BlockSpec((BLK, D1), lambda i: (i, 0)),
            pl.BlockSpec((BLK, D1), lambda i: (i, 0)),
        ],
        out_shape=[
            jax.ShapeDtypeStruct((N, D1), _f32),
            jax.ShapeDtypeStruct((N, D1), _f32),
        ],
    )(degAB, x)


def _tc2(s1AB, deg1, x, W1, b1):
    def body(s1_ref, deg1_ref, x_ref, w1_ref, b1_ref, out_ref):
        dcol = lax.rsqrt(deg1_ref[...][:, :1])
        xs = x_ref[...] * dcol
        s1 = s1_ref[0][:, :1] + s1_ref[1][:, :1]
        m1 = dcol * (s1 + xs)
        h1 = jnp.maximum(m1 * w1_ref[...][None, :] + b1_ref[...][None, :], 0.0)
        hs1 = h1 * dcol
        out_ref[0] = hs1[:, :D2]
        out_ref[1] = hs1[:, D2:]

    return pl.pallas_call(
        body,
        grid=(N // BLK,),
        in_specs=[
            pl.BlockSpec((NC, BLK, D1), lambda i: (0, i, 0)),
            pl.BlockSpec((BLK, D1), lambda i: (i, 0)),
            pl.BlockSpec((BLK, 1), lambda i: (i, 0)),
            pl.BlockSpec((256,), lambda i: (0,)),
            pl.BlockSpec((256,), lambda i: (0,)),
        ],
        out_specs=pl.BlockSpec((NC, BLK, D2), lambda i: (0, i, 0)),
        out_shape=jax.ShapeDtypeStruct((NC, N, D2), _f32),
    )(s1AB, deg1, x, W1, b1)


def _tc3(m2, hs1, deg1, W2, b2):
    def body(m_ref, hs_ref, deg1_ref, w_ref, b_ref, out_ref):
        dcol = lax.rsqrt(deg1_ref[...][:, :1])
        p = dcol * jnp.concatenate(
            [m_ref[0] + hs_ref[0], m_ref[1] + hs_ref[1]], axis=1)
        h = jnp.maximum(
            jnp.dot(p, w_ref[...], preferred_element_type=_f32, precision=_HI)
            + b_ref[...][None, :], 0.0)
        hs = h * dcol
        out_ref[0] = hs[:, :D2]
        out_ref[1] = hs[:, D2:]

    return pl.pallas_call(
        body,
        grid=(N // BLK,),
        in_specs=[
            pl.BlockSpec((NC, BLK, D2), lambda i: (0, i, 0)),
            pl.BlockSpec((NC, BLK, D2), lambda i: (0, i, 0)),
            pl.BlockSpec((BLK, D1), lambda i: (i, 0)),
            pl.BlockSpec((256, 256), lambda i: (0, 0)),
            pl.BlockSpec((256,), lambda i: (0,)),
        ],
        out_specs=pl.BlockSpec((NC, BLK, D2), lambda i: (0, i, 0)),
        out_shape=jax.ShapeDtypeStruct((NC, N, D2), _f32),
    )(m2, hs1, deg1, W2, b2)


def _ln_silu(z, g, b, eps=1e-5):
    m = jnp.mean(z, axis=-1, keepdims=True)
    v = jnp.mean((z - m) ** 2, axis=-1, keepdims=True)
    z = (z - m) * lax.rsqrt(v + eps) * g[None, :] + b[None, :]
    return z * jax.nn.sigmoid(z)


def _tc4(m3, hs2, deg1, vecp, Wvp, bv, W3, b3, WaggT, bagg,
         g1, be1, Wc1, bc1, g2, be2, Wc2, bc2, g3, be3):
    def body(m_ref, hs_ref, deg1_ref, vec_ref, wv_ref, bv_ref, w3_ref, b3_ref,
             wa_ref, ba_ref, g1_ref, be1_ref, wc1_ref, bc1_ref,
             g2_ref, be2_ref, wc2_ref, bc2_ref, g3_ref, be3_ref, out_ref):
        dcol = lax.rsqrt(deg1_ref[...][:, :1])
        p = dcol * jnp.concatenate(
            [m_ref[0] + hs_ref[0], m_ref[1] + hs_ref[1]], axis=1)
        h3 = jnp.maximum(
            jnp.dot(p, w3_ref[...], preferred_element_type=_f32, precision=_HI)
            + b3_ref[...][None, :], 0.0)
        agg = jnp.dot(h3, wa_ref[...], preferred_element_type=_f32,
                      precision=_HI) + ba_ref[...][None, :]
        hv = jnp.dot(vec_ref[...], wv_ref[...], preferred_element_type=_f32,
                     precision=_HI) + bv_ref[...][None, :]
        z = jnp.concatenate([agg, hv], axis=1)
        z = _ln_silu(z, g1_ref[...], be1_ref[...])
        z = jnp.dot(z, wc1_ref[...], preferred_element_type=_f32,
                    precision=_HI) + bc1_ref[...][None, :]
        z = _ln_silu(z, g2_ref[...], be2_ref[...])
        z = jnp.dot(z, wc2_ref[...], preferred_element_type=_f32,
                    precision=_HI) + bc2_ref[...][None, :]
        out_ref[...] = _ln_silu(z, g3_ref[...], be3_ref[...])

    full = lambda shape: pl.BlockSpec(shape, lambda i: tuple(0 for _ in shape))
    return pl.pallas_call(
        body,
        grid=(N // BLK,),
        in_specs=[
            pl.BlockSpec((NC, BLK, D2), lambda i: (0, i, 0)),
            pl.BlockSpec((NC, BLK, D2), lambda i: (0, i, 0)),
            pl.BlockSpec((BLK, D1), lambda i: (i, 0)),
            pl.BlockSpec((BLK, 8), lambda i: (i, 0)),
            full((8, 256)), full((256,)),
            full((256, 256)), full((256,)),
            full((256, 32)), full((32,)),
            full((288,)), full((288,)),
            full((288, 512)), full((512,)),
            full((512,)), full((512,)),
            full((512, 512)), full((512,)),
            full((512,)), full((512,)),
        ],
        out_specs=pl.BlockSpec((BLK, 512), lambda i: (i, 0)),
        out_shape=jax.ShapeDtypeStruct((N, 512), _f32),
    )(m3, hs2, deg1, vecp, Wvp, bv, W3, b3, WaggT, bagg,
      g1, be1, Wc1, bc1, g2, be2, Wc2, bc2, g3, be3)


def kernel(x, edge_index, batch_index, vector, W1, b1, W2, b2, W3, b3,
           Wagg, bagg, Wv, bv, g1, be1, Wc1, bc1, g2, be2, Wc2, bc2, g3, be3):
    del batch_index
    src = edge_index[0]
    dst = edge_index[1]
    pad = EPAD - E
    src2d = jnp.concatenate(
        [src, jnp.zeros((pad,), jnp.int32)]).reshape(NCHUNKS, CHUNK)
    dst2d = jnp.concatenate(
        [dst, jnp.full((pad,), TRASH, jnp.int32)]).reshape(NCHUNKS, CHUNK)

    degAB = _sc_deg(dst2d)
    deg1, xs16 = _tc1(degAB, x)
    s1AB = _sc_narrow(xs16, src2d, dst2d)
    hs1 = _tc2(s1AB, deg1, x, W1[0], b1)
    m2 = _sc_wide(hs1, src2d, dst2d)
    hs2 = _tc3(m2, hs1, deg1, W2, b2)
    m3 = _sc_wide(hs2, src2d, dst2d)
    vecp = jnp.pad(vector, ((0, 0), (0, 2)))
    Wvp = jnp.pad(Wv, ((0, 2), (0, 0)))
    return _tc4(m3, hs2, deg1, vecp, Wvp, bv, W3, b3, Wagg[:256], bagg,
                g1, be1, Wc1, bc1, g2, be2, Wc2, bc2, g3, be3)


# speedup vs baseline: 1.1740x; 1.0264x over previous
import functools

import jax
import jax.numpy as jnp
from jax import lax
from jax.experimental import pallas as pl
from jax.experimental.pallas import tpu as pltpu
from jax.experimental.pallas import tpu_sc as plsc

N = 10000
E = 160000
NC, NS, L = 2, 16, 16
CHUNK = 128
EPAD = NC * NS * CHUNK * 40
NCHUNKS = EPAD // CHUNK
CPW_HALF = NCHUNKS // (NC * NS)
CPW_FULL = NCHUNKS // NS
KB = 16
NB = CPW_FULL // KB
KB2 = 8
NB2 = CPW_HALF // KB2
TRASH = N
ROWS_A = 10240
D1 = 16
D2 = 128
BLK = 1000

_mesh = plsc.VectorSubcoreMesh(
    core_axis_name="c", subcore_axis_name="s", num_cores=NC, num_subcores=NS)

_f32 = jnp.float32
_HI = lax.Precision.DEFAULT
_sc_params = pltpu.CompilerParams(use_tc_tiling_on_sc=False)


def _fill_rows(buf, width, value):
    @pl.loop(0, CHUNK)
    def _(i):
        @pl.loop(0, width // L)
        def _(j):
            buf[i, pl.ds(j * L, L)] = jnp.full((L,), value, _f32)


def _zero_accum(zero_v, accum, s):
    @pl.loop(0, 5)
    def _(k):
        pltpu.sync_copy(zero_v, accum.at[pl.ds(s * 640 + k * CHUNK, CHUNK)])


def _sc_deg(dst2d):
    @functools.partial(
        pl.kernel,
        out_type=jax.ShapeDtypeStruct((NC, ROWS_A, D1), _f32),
        mesh=_mesh,
        scratch_types=[
            pltpu.VMEM((CPW_HALF, CHUNK), jnp.int32),
            pltpu.VMEM((CHUNK, D1), _f32),
            pltpu.VMEM_SHARED((ROWS_A, D1), _f32),
        ],
        compiler_params=_sc_params,
    )
    def k(dst_hbm, out_hbm, idx_v, ones_v, accum):
        c = lax.axis_index("c")
        s = lax.axis_index("s")
        wc = c * NS + s
        _fill_rows(ones_v, D1, 0.0)
        _zero_accum(ones_v, accum, s)
        _fill_rows(ones_v, D1, 1.0)
        pltpu.sync_copy(dst_hbm.at[pl.ds(wc * CPW_HALF, CPW_HALF)], idx_v)
        plsc.subcore_barrier()

        @pl.loop(0, CPW_HALF)
        def _(j):
            pltpu.sync_copy(ones_v, accum.at[idx_v.at[j]], add=True)

        plsc.subcore_barrier()
        pltpu.sync_copy(accum.at[pl.ds(s * 640, 640)],
                        out_hbm.at[c].at[pl.ds(s * 640, 640)])

    return k(dst2d)


def _sc_narrow(xs16, src2d, dst2d):
    @functools.partial(
        pl.kernel,
        out_type=jax.ShapeDtypeStruct((NC, ROWS_A, D1), _f32),
        mesh=_mesh,
        scratch_types=[
            pltpu.VMEM((CPW_HALF, CHUNK), jnp.int32),
            pltpu.VMEM((CPW_HALF, CHUNK), jnp.int32),
            pltpu.VMEM((CHUNK, D1), _f32),
            pltpu.VMEM((CHUNK, D1), _f32),
            pltpu.VMEM_SHARED((ROWS_A, D1), _f32),
            pltpu.SemaphoreType.DMA,
            pltpu.SemaphoreType.DMA,
        ],
        compiler_params=_sc_params,
    )
    def k(xs_hbm, src_hbm, dst_hbm, out_hbm, isrc_v, idst_v,
          rows_a, rows_b, accum, sem_a, sem_b):
        c = lax.axis_index("c")
        s = lax.axis_index("s")
        wc = c * NS + s
        bufs = (rows_a, rows_b)
        sems = (sem_a, sem_b)

        def g_start(j, b):
            pltpu.async_copy(xs_hbm.at[isrc_v.at[j]], bufs[b], sems[b])

        def step(j, i, last):
            b = i % 2
            if not (last and i == KB2 - 1):
                g_start(j + 1, 1 - b)
            pltpu.make_async_copy(
                xs_hbm.at[isrc_v.at[j]], bufs[b], sems[b]).wait()
            pltpu.sync_copy(bufs[b], accum.at[idst_v.at[j]], add=True)

        _fill_rows(rows_a, D1, 0.0)
        _zero_accum(rows_a, accum, s)
        pltpu.sync_copy(src_hbm.at[pl.ds(wc * CPW_HALF, CPW_HALF)], isrc_v)
        pltpu.sync_copy(dst_hbm.at[pl.ds(wc * CPW_HALF, CPW_HALF)], idst_v)
        plsc.subcore_barrier()

        g_start(0, 0)

        @pl.loop(0, NB2 - 1)
        def _(blk):
            for i in range(KB2):
                step(blk * KB2 + i, i, False)

        for i in range(KB2):
            step((NB2 - 1) * KB2 + i, i, True)

        plsc.subcore_barrier()
        pltpu.sync_copy(accum.at[pl.ds(s * 640, 640)],
                        out_hbm.at[c].at[pl.ds(s * 640, 640)])

    return k(xs16, src2d, dst2d)


def _sc_wide(hs, src2d, dst2d):
    @functools.partial(
        pl.kernel,
        out_type=jax.ShapeDtypeStruct((NC, ROWS_A, D2), _f32),
        mesh=_mesh,
        scratch_types=[
            pltpu.VMEM((KB, CHUNK), jnp.int32),
            pltpu.VMEM((KB, CHUNK), jnp.int32),
            pltpu.VMEM((CHUNK, D2), _f32),
            pltpu.VMEM((CHUNK, D2), _f32),
            pltpu.VMEM_SHARED((ROWS_A, D2), _f32),
            pltpu.SemaphoreType.DMA,
            pltpu.SemaphoreType.DMA,
        ],
    )
    def k(hs_hbm, src_hbm, dst_hbm, out_hbm, isrc_v, idst_v,
          rows_a, rows_b, accum, sem_a, sem_b):
        c = lax.axis_index("c")
        s = lax.axis_index("s")
        bufs = (rows_a, rows_b)
        sems = (sem_a, sem_b)
        _fill_rows(rows_a, D2, 0.0)
        _zero_accum(rows_a, accum, s)
        plsc.subcore_barrier()

        @pl.loop(0, NB)
        def _(blk):
            base = s * CPW_FULL + blk * KB
            pltpu.sync_copy(src_hbm.at[pl.ds(base, KB)], isrc_v)
            pltpu.sync_copy(dst_hbm.at[pl.ds(base, KB)], idst_v)
            pltpu.async_copy(hs_hbm.at[c].at[isrc_v.at[0]], bufs[0], sems[0])
            for i in range(KB):
                b = i % 2
                if i + 1 < KB:
                    pltpu.async_copy(hs_hbm.at[c].at[isrc_v.at[i + 1]],
                                     bufs[1 - b], sems[1 - b])
                pltpu.make_async_copy(
                    hs_hbm.at[c].at[isrc_v.at[i]], bufs[b], sems[b]).wait()
                pltpu.sync_copy(bufs[b], accum.at[idst_v.at[i]], add=True)

        plsc.subcore_barrier()
        pltpu.sync_copy(accum.at[pl.ds(s * 640, 640)],
                        out_hbm.at[c].at[pl.ds(s * 640, 640)])

    return k(hs, src2d, dst2d)


def _tc1(degAB, x):
    def body(dab_ref, x_ref, deg1_ref, xs16_ref):
        d = dab_ref[0] + dab_ref[1] + 1.0
        deg1_ref[...] = d
        xs16_ref[...] = x_ref[...] * lax.rsqrt(d)

    return pl.pallas_call(
        body,
        grid=(N // BLK,),
        in_specs=[
            pl.BlockSpec((NC, BLK, D1), lambda i: (0, i, 0)),
            pl.BlockSpec((BLK, 1), lambda i: (i, 0)),
        ],
        out_specs=[
            pl.BlockSpec((BLK, D1), lambda i: (i, 0)),
            pl.BlockSpec((BLK, D1), lambda i: (i, 0)),
        ],
        out_shape=[
            jax.ShapeDtypeStruct((N, D1), _f32),
            jax.ShapeDtypeStruct((N, D1), _f32),
        ],
    )(degAB, x)


def _tc2(s1AB, deg1, x, W1, b1):
    def body(s1_ref, deg1_ref, x_ref, w1_ref, b1_ref, out_ref):
        dcol = lax.rsqrt(deg1_ref[...][:, :1])
        xs = x_ref[...] * dcol
        s1 = s1_ref[0][:, :1] + s1_ref[1][:, :1]
        m1 = dcol * (s1 + xs)
        h1 = jnp.maximum(m1 * w1_ref[...][None, :] + b1_ref[...][None, :], 0.0)
        hs1 = h1 * dcol
        out_ref[0] = hs1[:, :D2]
        out_ref[1] = hs1[:, D2:]

    return pl.pallas_call(
        body,
        grid=(N // BLK,),
        in_specs=[
            pl.BlockSpec((NC, BLK, D1), lambda i: (0, i, 0)),
            pl.BlockSpec((BLK, D1), lambda i: (i, 0)),
            pl.BlockSpec((BLK, 1), lambda i: (i, 0)),
            pl.BlockSpec((256,), lambda i: (0,)),
            pl.BlockSpec((256,), lambda i: (0,)),
        ],
        out_specs=pl.BlockSpec((NC, BLK, D2), lambda i: (0, i, 0)),
        out_shape=jax.ShapeDtypeStruct((NC, N, D2), _f32),
    )(s1AB, deg1, x, W1, b1)


def _tc3(m2, hs1, deg1, W2, b2):
    def body(m_ref, hs_ref, deg1_ref, w_ref, b_ref, out_ref):
        dcol = lax.rsqrt(deg1_ref[...][:, :1])
        p = dcol * jnp.concatenate(
            [m_ref[0] + hs_ref[0], m_ref[1] + hs_ref[1]], axis=1)
        h = jnp.maximum(
            jnp.dot(p, w_ref[...], preferred_element_type=_f32, precision=_HI)
            + b_ref[...][None, :], 0.0)
        hs = h * dcol
        out_ref[0] = hs[:, :D2]
        out_ref[1] = hs[:, D2:]

    return pl.pallas_call(
        body,
        grid=(N // BLK,),
        in_specs=[
            pl.BlockSpec((NC, BLK, D2), lambda i: (0, i, 0)),
            pl.BlockSpec((NC, BLK, D2), lambda i: (0, i, 0)),
            pl.BlockSpec((BLK, D1), lambda i: (i, 0)),
            pl.BlockSpec((256, 256), lambda i: (0, 0)),
            pl.BlockSpec((256,), lambda i: (0,)),
        ],
        out_specs=pl.BlockSpec((NC, BLK, D2), lambda i: (0, i, 0)),
        out_shape=jax.ShapeDtypeStruct((NC, N, D2), _f32),
    )(m2, hs1, deg1, W2, b2)


def _ln_silu(z, g, b, eps=1e-5):
    m = jnp.mean(z, axis=-1, keepdims=True)
    v = jnp.mean((z - m) ** 2, axis=-1, keepdims=True)
    z = (z - m) * lax.rsqrt(v + eps) * g[None, :] + b[None, :]
    return z * jax.nn.sigmoid(z)


def _tc4(m3, hs2, deg1, vecp, Wvp, bv, W3, b3, WaggT, bagg,
         g1, be1, Wc1, bc1, g2, be2, Wc2, bc2, g3, be3):
    def body(m_ref, hs_ref, deg1_ref, vec_ref, wv_ref, bv_ref, w3_ref, b3_ref,
             wa_ref, ba_ref, g1_ref, be1_ref, wc1_ref, bc1_ref,
             g2_ref, be2_ref, wc2_ref, bc2_ref, g3_ref, be3_ref, out_ref):
        dcol = lax.rsqrt(deg1_ref[...][:, :1])
        p = dcol * jnp.concatenate(
            [m_ref[0] + hs_ref[0], m_ref[1] + hs_ref[1]], axis=1)
        h3 = jnp.maximum(
            jnp.dot(p, w3_ref[...], preferred_element_type=_f32, precision=_HI)
            + b3_ref[...][None, :], 0.0)
        agg = jnp.dot(h3, wa_ref[...], preferred_element_type=_f32,
                      precision=_HI) + ba_ref[...][None, :]
        hv = jnp.dot(vec_ref[...], wv_ref[...], preferred_element_type=_f32,
                     precision=_HI) + bv_ref[...][None, :]
        z = jnp.concatenate([agg, hv], axis=1)
        z = _ln_silu(z, g1_ref[...], be1_ref[...])
        z = jnp.dot(z, wc1_ref[...], preferred_element_type=_f32,
                    precision=_HI) + bc1_ref[...][None, :]
        z = _ln_silu(z, g2_ref[...], be2_ref[...])
        z = jnp.dot(z, wc2_ref[...], preferred_element_type=_f32,
                    precision=_HI) + bc2_ref[...][None, :]
        out_ref[...] = _ln_silu(z, g3_ref[...], be3_ref[...])

    full = lambda shape: pl.BlockSpec(shape, lambda i: tuple(0 for _ in shape))
    return pl.pallas_call(
        body,
        grid=(N // BLK,),
        in_specs=[
            pl.BlockSpec((NC, BLK, D2), lambda i: (0, i, 0)),
            pl.BlockSpec((NC, BLK, D2), lambda i: (0, i, 0)),
            pl.BlockSpec((BLK, D1), lambda i: (i, 0)),
            pl.BlockSpec((BLK, 8), lambda i: (i, 0)),
            full((8, 256)), full((256,)),
            full((256, 256)), full((256,)),
            full((256, 32)), full((32,)),
            full((288,)), full((288,)),
            full((288, 512)), full((512,)),
            full((512,)), full((512,)),
            full((512, 512)), full((512,)),
            full((512,)), full((512,)),
        ],
        out_specs=pl.BlockSpec((BLK, 512), lambda i: (i, 0)),
        out_shape=jax.ShapeDtypeStruct((N, 512), _f32),
    )(m3, hs2, deg1, vecp, Wvp, bv, W3, b3, WaggT, bagg,
      g1, be1, Wc1, bc1, g2, be2, Wc2, bc2, g3, be3)


def kernel(x, edge_index, batch_index, vector, W1, b1, W2, b2, W3, b3,
           Wagg, bagg, Wv, bv, g1, be1, Wc1, bc1, g2, be2, Wc2, bc2, g3, be3):
    del batch_index
    src = edge_index[0]
    dst = edge_index[1]
    pad = EPAD - E
    src2d = jnp.concatenate(
        [src, jnp.zeros((pad,), jnp.int32)]).reshape(NCHUNKS, CHUNK)
    dst2d = jnp.concatenate(
        [dst, jnp.full((pad,), TRASH, jnp.int32)]).reshape(NCHUNKS, CHUNK)

    degAB = _sc_deg(dst2d)
    deg1, xs16 = _tc1(degAB, x)
    s1AB = _sc_narrow(xs16, src2d, dst2d)
    hs1 = _tc2(s1AB, deg1, x, W1[0], b1)
    m2 = _sc_wide(hs1, src2d, dst2d)
    hs2 = _tc3(m2, hs1, deg1, W2, b2)
    m3 = _sc_wide(hs2, src2d, dst2d)
    vecp = jnp.pad(vector, ((0, 0), (0, 2)))
    Wvp = jnp.pad(Wv, ((0, 2), (0, 0)))
    return _tc4(m3, hs2, deg1, vecp, Wvp, bv, W3, b3, Wagg[:256], bagg,
                g1, be1, Wc1, bc1, g2, be2, Wc2, bc2, g3, be3)


# speedup vs baseline: 2.1498x; 1.8312x over previous
import functools

import jax
import jax.numpy as jnp
from jax import lax
from jax.experimental import pallas as pl
from jax.experimental.pallas import tpu as pltpu
from jax.experimental.pallas import tpu_sc as plsc

N = 10000
E = 160000
NC, NS, L = 2, 16, 16
CHUNK = 125
NCHUNKS = E // CHUNK
CPW_HALF = NCHUNKS // (NC * NS)
CPW_FULL = NCHUNKS // NS
KB = 16
NB = CPW_FULL // KB
KB2 = 8
NB2 = CPW_HALF // KB2
ROWS_A = 10240
D1 = 16
D2 = 128
BLK = 1000

_mesh = plsc.VectorSubcoreMesh(
    core_axis_name="c", subcore_axis_name="s", num_cores=NC, num_subcores=NS)

_f32 = jnp.float32
_HI = lax.Precision.DEFAULT
_sc_params = pltpu.CompilerParams(use_tc_tiling_on_sc=False)


def _fill_rows(buf, width, value):
    @pl.loop(0, CHUNK)
    def _(i):
        @pl.loop(0, width // L)
        def _(j):
            buf[i, pl.ds(j * L, L)] = jnp.full((L,), value, _f32)


def _zero_accum(zero_v, accum, s):
    @pl.loop(0, 8)
    def _(k):
        pltpu.sync_copy(zero_v.at[pl.ds(0, 80)],
                        accum.at[pl.ds(s * 640 + k * 80, 80)])


def _sc_deg(dst2d):
    @functools.partial(
        pl.kernel,
        out_type=jax.ShapeDtypeStruct((NC, ROWS_A, D1), _f32),
        mesh=_mesh,
        scratch_types=[
            pltpu.VMEM((CPW_HALF, CHUNK), jnp.int32),
            pltpu.VMEM((CHUNK, D1), _f32),
            pltpu.VMEM_SHARED((ROWS_A, D1), _f32),
        ],
        compiler_params=_sc_params,
    )
    def k(dst_hbm, out_hbm, idx_v, ones_v, accum):
        c = lax.axis_index("c")
        s = lax.axis_index("s")
        wc = c * NS + s
        _fill_rows(ones_v, D1, 0.0)
        _zero_accum(ones_v, accum, s)
        _fill_rows(ones_v, D1, 1.0)
        pltpu.sync_copy(dst_hbm.at[pl.ds(wc * CPW_HALF, CPW_HALF)], idx_v)
        plsc.subcore_barrier()

        @pl.loop(0, CPW_HALF)
        def _(j):
            pltpu.sync_copy(ones_v, accum.at[idx_v.at[j]], add=True)

        plsc.subcore_barrier()
        pltpu.sync_copy(accum.at[pl.ds(s * 640, 640)],
                        out_hbm.at[c].at[pl.ds(s * 640, 640)])

    return k(dst2d)


def _sc_narrow(xs16, src2d, dst2d):
    @functools.partial(
        pl.kernel,
        out_type=jax.ShapeDtypeStruct((NC, ROWS_A, D1), _f32),
        mesh=_mesh,
        scratch_types=[
            pltpu.VMEM((CPW_HALF, CHUNK), jnp.int32),
            pltpu.VMEM((CPW_HALF, CHUNK), jnp.int32),
            pltpu.VMEM((CHUNK, D1), _f32),
            pltpu.VMEM((CHUNK, D1), _f32),
            pltpu.VMEM_SHARED((ROWS_A, D1), _f32),
            pltpu.SemaphoreType.DMA,
            pltpu.SemaphoreType.DMA,
        ],
        compiler_params=_sc_params,
    )
    def k(xs_hbm, src_hbm, dst_hbm, out_hbm, isrc_v, idst_v,
          rows_a, rows_b, accum, sem_a, sem_b):
        c = lax.axis_index("c")
        s = lax.axis_index("s")
        wc = c * NS + s
        bufs = (rows_a, rows_b)
        sems = (sem_a, sem_b)

        def g_start(j, b):
            pltpu.async_copy(xs_hbm.at[isrc_v.at[j]], bufs[b], sems[b])

        def step(j, i, last):
            b = i % 2
            if not (last and i == KB2 - 1):
                g_start(j + 1, 1 - b)
            pltpu.make_async_copy(
                xs_hbm.at[isrc_v.at[j]], bufs[b], sems[b]).wait()
            pltpu.sync_copy(bufs[b], accum.at[idst_v.at[j]], add=True)

        _fill_rows(rows_a, D1, 0.0)
        _zero_accum(rows_a, accum, s)
        pltpu.sync_copy(src_hbm.at[pl.ds(wc * CPW_HALF, CPW_HALF)], isrc_v)
        pltpu.sync_copy(dst_hbm.at[pl.ds(wc * CPW_HALF, CPW_HALF)], idst_v)
        plsc.subcore_barrier()

        g_start(0, 0)

        @pl.loop(0, NB2 - 1)
        def _(blk):
            for i in range(KB2):
                step(blk * KB2 + i, i, False)

        for i in range(KB2):
            step((NB2 - 1) * KB2 + i, i, True)

        plsc.subcore_barrier()
        pltpu.sync_copy(accum.at[pl.ds(s * 640, 640)],
                        out_hbm.at[c].at[pl.ds(s * 640, 640)])

    return k(xs16, src2d, dst2d)


def _sc_wide(hs, src2d, dst2d):
    @functools.partial(
        pl.kernel,
        out_type=jax.ShapeDtypeStruct((NC, ROWS_A, D2), _f32),
        mesh=_mesh,
        scratch_types=[
            pltpu.VMEM((KB, CHUNK), jnp.int32),
            pltpu.VMEM((KB, CHUNK), jnp.int32),
            pltpu.VMEM((CHUNK, D2), _f32),
            pltpu.VMEM((CHUNK, D2), _f32),
            pltpu.VMEM_SHARED((ROWS_A, D2), _f32),
            pltpu.SemaphoreType.DMA,
            pltpu.SemaphoreType.DMA,
        ],
    )
    def k(hs_hbm, src_hbm, dst_hbm, out_hbm, isrc_v, idst_v,
          rows_a, rows_b, accum, sem_a, sem_b):
        c = lax.axis_index("c")
        s = lax.axis_index("s")
        bufs = (rows_a, rows_b)
        sems = (sem_a, sem_b)
        _fill_rows(rows_a, D2, 0.0)
        _zero_accum(rows_a, accum, s)
        plsc.subcore_barrier()

        @pl.loop(0, NB)
        def _(blk):
            base = s * CPW_FULL + blk * KB
            pltpu.sync_copy(src_hbm.at[pl.ds(base, KB)], isrc_v)
            pltpu.sync_copy(dst_hbm.at[pl.ds(base, KB)], idst_v)
            pltpu.async_copy(hs_hbm.at[c].at[isrc_v.at[0]], bufs[0], sems[0])
            for i in range(KB):
                b = i % 2
                if i + 1 < KB:
                    pltpu.async_copy(hs_hbm.at[c].at[isrc_v.at[i + 1]],
                                     bufs[1 - b], sems[1 - b])
                pltpu.make_async_copy(
                    hs_hbm.at[c].at[isrc_v.at[i]], bufs[b], sems[b]).wait()
                pltpu.sync_copy(bufs[b], accum.at[idst_v.at[i]], add=True)

        plsc.subcore_barrier()
        pltpu.sync_copy(accum.at[pl.ds(s * 640, 640)],
                        out_hbm.at[c].at[pl.ds(s * 640, 640)])

    return k(hs, src2d, dst2d)


def _tc1(degAB, x):
    def body(dab_ref, x_ref, deg1_ref, xs16_ref):
        d = dab_ref[0] + dab_ref[1] + 1.0
        deg1_ref[...] = d
        xs16_ref[...] = x_ref[...] * lax.rsqrt(d)

    return pl.pallas_call(
        body,
        grid=(N // BLK,),
        in_specs=[
            pl.BlockSpec((NC, BLK, D1), lambda i: (0, i, 0)),
            pl.BlockSpec((BLK, 1), lambda i: (i, 0)),
        ],
        out_specs=[
            pl.BlockSpec((BLK, D1), lambda i: (i, 0)),
            pl.BlockSpec((BLK, D1), lambda i: (i, 0)),
        ],
        out_shape=[
            jax.ShapeDtypeStruct((N, D1), _f32),
            jax.ShapeDtypeStruct((N, D1), _f32),
        ],
    )(degAB, x)


def _tc2(s1AB, deg1, x, W1, b1):
    def body(s1_ref, deg1_ref, x_ref, w1_ref, b1_ref, out_ref):
        dcol = lax.rsqrt(deg1_ref[...][:, :1])
        xs = x_ref[...] * dcol
        s1 = s1_ref[0][:, :1] + s1_ref[1][:, :1]
        m1 = dcol * (s1 + xs)
        h1 = jnp.maximum(m1 * w1_ref[...][None, :] + b1_ref[...][None, :], 0.0)
        hs1 = h1 * dcol
        out_ref[0] = hs1[:, :D2]
        out_ref[1] = hs1[:, D2:]

    return pl.pallas_call(
        body,
        grid=(N // BLK,),
        in_specs=[
            pl.BlockSpec((NC, BLK, D1), lambda i: (0, i, 0)),
            pl.BlockSpec((BLK, D1), lambda i: (i, 0)),
            pl.BlockSpec((BLK, 1), lambda i: (i, 0)),
            pl.BlockSpec((256,), lambda i: (0,)),
            pl.BlockSpec((256,), lambda i: (0,)),
        ],
        out_specs=pl.BlockSpec((NC, BLK, D2), lambda i: (0, i, 0)),
        out_shape=jax.ShapeDtypeStruct((NC, N, D2), _f32),
    )(s1AB, deg1, x, W1, b1)


def _tc3(m2, hs1, deg1, W2, b2):
    def body(m_ref, hs_ref, deg1_ref, w_ref, b_ref, out_ref):
        dcol = lax.rsqrt(deg1_ref[...][:, :1])
        p = dcol * jnp.concatenate(
            [m_ref[0] + hs_ref[0], m_ref[1] + hs_ref[1]], axis=1)
        h = jnp.maximum(
            jnp.dot(p, w_ref[...], preferred_element_type=_f32, precision=_HI)
            + b_ref[...][None, :], 0.0)
        hs = h * dcol
        out_ref[0] = hs[:, :D2]
        out_ref[1] = hs[:, D2:]

    return pl.pallas_call(
        body,
        grid=(N // BLK,),
        in_specs=[
            pl.BlockSpec((NC, BLK, D2), lambda i: (0, i, 0)),
            pl.BlockSpec((NC, BLK, D2), lambda i: (0, i, 0)),
            pl.BlockSpec((BLK, D1), lambda i: (i, 0)),
            pl.BlockSpec((256, 256), lambda i: (0, 0)),
            pl.BlockSpec((256,), lambda i: (0,)),
        ],
        out_specs=pl.BlockSpec((NC, BLK, D2), lambda i: (0, i, 0)),
        out_shape=jax.ShapeDtypeStruct((NC, N, D2), _f32),
    )(m2, hs1, deg1, W2, b2)


def _ln_silu(z, g, b, eps=1e-5):
    m = jnp.mean(z, axis=-1, keepdims=True)
    v = jnp.mean((z - m) ** 2, axis=-1, keepdims=True)
    z = (z - m) * lax.rsqrt(v + eps) * g[None, :] + b[None, :]
    return z * jax.nn.sigmoid(z)


def _tc4(m3, hs2, deg1, vecp, Wvp, bv, W3, b3, WaggT, bagg,
         g1, be1, Wc1, bc1, g2, be2, Wc2, bc2, g3, be3):
    def body(m_ref, hs_ref, deg1_ref, vec_ref, wv_ref, bv_ref, w3_ref, b3_ref,
             wa_ref, ba_ref, g1_ref, be1_ref, wc1_ref, bc1_ref,
             g2_ref, be2_ref, wc2_ref, bc2_ref, g3_ref, be3_ref, out_ref):
        dcol = lax.rsqrt(deg1_ref[...][:, :1])
        p = dcol * jnp.concatenate(
            [m_ref[0] + hs_ref[0], m_ref[1] + hs_ref[1]], axis=1)
        h3 = jnp.maximum(
            jnp.dot(p, w3_ref[...], preferred_element_type=_f32, precision=_HI)
            + b3_ref[...][None, :], 0.0)
        agg = jnp.dot(h3, wa_ref[...], preferred_element_type=_f32,
                      precision=_HI) + ba_ref[...][None, :]
        hv = jnp.dot(vec_ref[...], wv_ref[...], preferred_element_type=_f32,
                     precision=_HI) + bv_ref[...][None, :]
        z = jnp.concatenate([agg, hv], axis=1)
        z = _ln_silu(z, g1_ref[...], be1_ref[...])
        z = jnp.dot(z, wc1_ref[...], preferred_element_type=_f32,
                    precision=_HI) + bc1_ref[...][None, :]
        z = _ln_silu(z, g2_ref[...], be2_ref[...])
        z = jnp.dot(z, wc2_ref[...], preferred_element_type=_f32,
                    precision=_HI) + bc2_ref[...][None, :]
        out_ref[...] = _ln_silu(z, g3_ref[...], be3_ref[...])

    full = lambda shape: pl.BlockSpec(shape, lambda i: tuple(0 for _ in shape))
    return pl.pallas_call(
        body,
        grid=(N // BLK,),
        in_specs=[
            pl.BlockSpec((NC, BLK, D2), lambda i: (0, i, 0)),
            pl.BlockSpec((NC, BLK, D2), lambda i: (0, i, 0)),
            pl.BlockSpec((BLK, D1), lambda i: (i, 0)),
            pl.BlockSpec((BLK, 8), lambda i: (i, 0)),
            full((8, 256)), full((256,)),
            full((256, 256)), full((256,)),
            full((256, 32)), full((32,)),
            full((288,)), full((288,)),
            full((288, 512)), full((512,)),
            full((512,)), full((512,)),
            full((512, 512)), full((512,)),
            full((512,)), full((512,)),
        ],
        out_specs=pl.BlockSpec((BLK, 512), lambda i: (i, 0)),
        out_shape=jax.ShapeDtypeStruct((N, 512), _f32),
    )(m3, hs2, deg1, vecp, Wvp, bv, W3, b3, WaggT, bagg,
      g1, be1, Wc1, bc1, g2, be2, Wc2, bc2, g3, be3)


def kernel(x, edge_index, batch_index, vector, W1, b1, W2, b2, W3, b3,
           Wagg, bagg, Wv, bv, g1, be1, Wc1, bc1, g2, be2, Wc2, bc2, g3, be3):
    del batch_index
    src2d = edge_index[0].reshape(NCHUNKS, CHUNK)
    dst2d = edge_index[1].reshape(NCHUNKS, CHUNK)

    degAB = _sc_deg(dst2d)
    deg1, xs16 = _tc1(degAB, x)
    s1AB = _sc_narrow(xs16, src2d, dst2d)
    hs1 = _tc2(s1AB, deg1, x, W1[0], b1)
    m2 = _sc_wide(hs1, src2d, dst2d)
    hs2 = _tc3(m2, hs1, deg1, W2, b2)
    m3 = _sc_wide(hs2, src2d, dst2d)
    vecp = jnp.pad(vector, ((0, 0), (0, 2)))
    Wvp = jnp.pad(Wv, ((0, 2), (0, 0)))
    return _tc4(m3, hs2, deg1, vecp, Wvp, bv, W3, b3, Wagg[:256], bagg,
                g1, be1, Wc1, bc1, g2, be2, Wc2, bc2, g3, be3)


# speedup vs baseline: 2.1801x; 1.0141x over previous
import functools

import jax
import jax.numpy as jnp
from jax import lax
from jax.experimental import pallas as pl
from jax.experimental.pallas import tpu as pltpu
from jax.experimental.pallas import tpu_sc as plsc

N = 10000
E = 160000
NC, NS, L = 2, 16, 16
CHUNK = 125
NCHUNKS = E // CHUNK
CPW_HALF = NCHUNKS // (NC * NS)
CPW_FULL = NCHUNKS // NS
KB = 16
NB = CPW_FULL // KB
KB2 = 8
NB2 = CPW_HALF // KB2
ROWS_A = 10240
D1 = 16
D2 = 128
BLK = 2000

_mesh = plsc.VectorSubcoreMesh(
    core_axis_name="c", subcore_axis_name="s", num_cores=NC, num_subcores=NS)

_f32 = jnp.float32
_HI = lax.Precision.DEFAULT
_sc_params = pltpu.CompilerParams(use_tc_tiling_on_sc=False)


def _fill_rows(buf, width, value):
    @pl.loop(0, CHUNK)
    def _(i):
        @pl.loop(0, width // L)
        def _(j):
            buf[i, pl.ds(j * L, L)] = jnp.full((L,), value, _f32)


def _zero_accum(zero_v, accum, s):
    @pl.loop(0, 8)
    def _(k):
        pltpu.sync_copy(zero_v.at[pl.ds(0, 80)],
                        accum.at[pl.ds(s * 640 + k * 80, 80)])


def _sc_deg(dst2d):
    @functools.partial(
        pl.kernel,
        out_type=jax.ShapeDtypeStruct((NC, ROWS_A, D1), _f32),
        mesh=_mesh,
        scratch_types=[
            pltpu.VMEM((CPW_HALF, CHUNK), jnp.int32),
            pltpu.VMEM((CHUNK, D1), _f32),
            pltpu.VMEM_SHARED((ROWS_A, D1), _f32),
        ],
        compiler_params=_sc_params,
    )
    def k(dst_hbm, out_hbm, idx_v, ones_v, accum):
        c = lax.axis_index("c")
        s = lax.axis_index("s")
        wc = c * NS + s
        _fill_rows(ones_v, D1, 0.0)
        _zero_accum(ones_v, accum, s)
        _fill_rows(ones_v, D1, 1.0)
        pltpu.sync_copy(dst_hbm.at[pl.ds(wc * CPW_HALF, CPW_HALF)], idx_v)
        plsc.subcore_barrier()

        @pl.loop(0, CPW_HALF)
        def _(j):
            pltpu.sync_copy(ones_v, accum.at[idx_v.at[j]], add=True)

        plsc.subcore_barrier()
        pltpu.sync_copy(accum.at[pl.ds(s * 640, 640)],
                        out_hbm.at[c].at[pl.ds(s * 640, 640)])

    return k(dst2d)


def _sc_narrow(xs16, src2d, dst2d):
    @functools.partial(
        pl.kernel,
        out_type=jax.ShapeDtypeStruct((NC, ROWS_A, D1), _f32),
        mesh=_mesh,
        scratch_types=[
            pltpu.VMEM((CPW_HALF, CHUNK), jnp.int32),
            pltpu.VMEM((CPW_HALF, CHUNK), jnp.int32),
            pltpu.VMEM((CHUNK, D1), _f32),
            pltpu.VMEM((CHUNK, D1), _f32),
            pltpu.VMEM_SHARED((ROWS_A, D1), _f32),
            pltpu.SemaphoreType.DMA,
            pltpu.SemaphoreType.DMA,
        ],
        compiler_params=_sc_params,
    )
    def k(xs_hbm, src_hbm, dst_hbm, out_hbm, isrc_v, idst_v,
          rows_a, rows_b, accum, sem_a, sem_b):
        c = lax.axis_index("c")
        s = lax.axis_index("s")
        wc = c * NS + s
        bufs = (rows_a, rows_b)
        sems = (sem_a, sem_b)

        def g_start(j, b):
            pltpu.async_copy(xs_hbm.at[isrc_v.at[j]], bufs[b], sems[b])

        def step(j, i, last):
            b = i % 2
            if not (last and i == KB2 - 1):
                g_start(j + 1, 1 - b)
            pltpu.make_async_copy(
                xs_hbm.at[isrc_v.at[j]], bufs[b], sems[b]).wait()
            pltpu.sync_copy(bufs[b], accum.at[idst_v.at[j]], add=True)

        _fill_rows(rows_a, D1, 0.0)
        _zero_accum(rows_a, accum, s)
        pltpu.sync_copy(src_hbm.at[pl.ds(wc * CPW_HALF, CPW_HALF)], isrc_v)
        pltpu.sync_copy(dst_hbm.at[pl.ds(wc * CPW_HALF, CPW_HALF)], idst_v)
        plsc.subcore_barrier()

        g_start(0, 0)

        @pl.loop(0, NB2 - 1)
        def _(blk):
            for i in range(KB2):
                step(blk * KB2 + i, i, False)

        for i in range(KB2):
            step((NB2 - 1) * KB2 + i, i, True)

        plsc.subcore_barrier()
        pltpu.sync_copy(accum.at[pl.ds(s * 640, 640)],
                        out_hbm.at[c].at[pl.ds(s * 640, 640)])

    return k(xs16, src2d, dst2d)


def _sc_wide(hs, src2d, dst2d):
    @functools.partial(
        pl.kernel,
        out_type=jax.ShapeDtypeStruct((NC, ROWS_A, D2), _f32),
        mesh=_mesh,
        scratch_types=[
            pltpu.VMEM((KB, CHUNK), jnp.int32),
            pltpu.VMEM((KB, CHUNK), jnp.int32),
            pltpu.VMEM((CHUNK, D2), _f32),
            pltpu.VMEM((CHUNK, D2), _f32),
            pltpu.VMEM_SHARED((ROWS_A, D2), _f32),
            pltpu.SemaphoreType.DMA,
            pltpu.SemaphoreType.DMA,
        ],
    )
    def k(hs_hbm, src_hbm, dst_hbm, out_hbm, isrc_v, idst_v,
          rows_a, rows_b, accum, sem_a, sem_b):
        c = lax.axis_index("c")
        s = lax.axis_index("s")
        bufs = (rows_a, rows_b)
        sems = (sem_a, sem_b)
        _fill_rows(rows_a, D2, 0.0)
        _zero_accum(rows_a, accum, s)
        plsc.subcore_barrier()

        @pl.loop(0, NB)
        def _(blk):
            base = s * CPW_FULL + blk * KB
            pltpu.sync_copy(src_hbm.at[pl.ds(base, KB)], isrc_v)
            pltpu.sync_copy(dst_hbm.at[pl.ds(base, KB)], idst_v)
            pltpu.async_copy(hs_hbm.at[c].at[isrc_v.at[0]], bufs[0], sems[0])
            for i in range(KB):
                b = i % 2
                if i + 1 < KB:
                    pltpu.async_copy(hs_hbm.at[c].at[isrc_v.at[i + 1]],
                                     bufs[1 - b], sems[1 - b])
                pltpu.make_async_copy(
                    hs_hbm.at[c].at[isrc_v.at[i]], bufs[b], sems[b]).wait()
                pltpu.sync_copy(bufs[b], accum.at[idst_v.at[i]], add=True)

        plsc.subcore_barrier()
        pltpu.sync_copy(accum.at[pl.ds(s * 640, 640)],
                        out_hbm.at[c].at[pl.ds(s * 640, 640)])

    return k(hs, src2d, dst2d)


def _tc1(degAB, x):
    def body(dab_ref, x_ref, deg1_ref, xs16_ref):
        d = dab_ref[0] + dab_ref[1] + 1.0
        deg1_ref[...] = d
        xs16_ref[...] = x_ref[...] * lax.rsqrt(d)

    return pl.pallas_call(
        body,
        grid=(N // BLK,),
        in_specs=[
            pl.BlockSpec((NC, BLK, D1), lambda i: (0, i, 0)),
            pl.BlockSpec((BLK, 1), lambda i: (i, 0)),
        ],
        out_specs=[
            pl.BlockSpec((BLK, D1), lambda i: (i, 0)),
            pl.BlockSpec((BLK, D1), lambda i: (i, 0)),
        ],
        out_shape=[
            jax.ShapeDtypeStruct((N, D1), _f32),
            jax.ShapeDtypeStruct((N, D1), _f32),
        ],
    )(degAB, x)


def _tc2(s1AB, deg1, x, W1, b1):
    def body(s1_ref, deg1_ref, x_ref, w1_ref, b1_ref, out_ref):
        dcol = lax.rsqrt(deg1_ref[...][:, :1])
        xs = x_ref[...] * dcol
        s1 = s1_ref[0][:, :1] + s1_ref[1][:, :1]
        m1 = dcol * (s1 + xs)
        h1 = jnp.maximum(m1 * w1_ref[...][None, :] + b1_ref[...][None, :], 0.0)
        hs1 = h1 * dcol
        out_ref[0] = hs1[:, :D2]
        out_ref[1] = hs1[:, D2:]

    return pl.pallas_call(
        body,
        grid=(N // BLK,),
        in_specs=[
            pl.BlockSpec((NC, BLK, D1), lambda i: (0, i, 0)),
            pl.BlockSpec((BLK, D1), lambda i: (i, 0)),
            pl.BlockSpec((BLK, 1), lambda i: (i, 0)),
            pl.BlockSpec((256,), lambda i: (0,)),
            pl.BlockSpec((256,), lambda i: (0,)),
        ],
        out_specs=pl.BlockSpec((NC, BLK, D2), lambda i: (0, i, 0)),
        out_shape=jax.ShapeDtypeStruct((NC, N, D2), _f32),
    )(s1AB, deg1, x, W1, b1)


def _tc3(m2, hs1, deg1, W2, b2):
    def body(m_ref, hs_ref, deg1_ref, w_ref, b_ref, out_ref):
        dcol = lax.rsqrt(deg1_ref[...][:, :1])
        p = dcol * jnp.concatenate(
            [m_ref[0] + hs_ref[0], m_ref[1] + hs_ref[1]], axis=1)
        h = jnp.maximum(
            jnp.dot(p, w_ref[...], preferred_element_type=_f32, precision=_HI)
            + b_ref[...][None, :], 0.0)
        hs = h * dcol
        out_ref[0] = hs[:, :D2]
        out_ref[1] = hs[:, D2:]

    return pl.pallas_call(
        body,
        grid=(N // BLK,),
        in_specs=[
            pl.BlockSpec((NC, BLK, D2), lambda i: (0, i, 0)),
            pl.BlockSpec((NC, BLK, D2), lambda i: (0, i, 0)),
            pl.BlockSpec((BLK, D1), lambda i: (i, 0)),
            pl.BlockSpec((256, 256), lambda i: (0, 0)),
            pl.BlockSpec((256,), lambda i: (0,)),
        ],
        out_specs=pl.BlockSpec((NC, BLK, D2), lambda i: (0, i, 0)),
        out_shape=jax.ShapeDtypeStruct((NC, N, D2), _f32),
    )(m2, hs1, deg1, W2, b2)


def _ln_silu(z, g, b, eps=1e-5):
    m = jnp.mean(z, axis=-1, keepdims=True)
    v = jnp.mean((z - m) ** 2, axis=-1, keepdims=True)
    z = (z - m) * lax.rsqrt(v + eps) * g[None, :] + b[None, :]
    return z * jax.nn.sigmoid(z)


def _tc4(m3, hs2, deg1, vecp, Wvp, bv, W3, b3, WaggT, bagg,
         g1, be1, Wc1, bc1, g2, be2, Wc2, bc2, g3, be3):
    def body(m_ref, hs_ref, deg1_ref, vec_ref, wv_ref, bv_ref, w3_ref, b3_ref,
             wa_ref, ba_ref, g1_ref, be1_ref, wc1_ref, bc1_ref,
             g2_ref, be2_ref, wc2_ref, bc2_ref, g3_ref, be3_ref, out_ref):
        dcol = lax.rsqrt(deg1_ref[...][:, :1])
        p = dcol * jnp.concatenate(
            [m_ref[0] + hs_ref[0], m_ref[1] + hs_ref[1]], axis=1)
        h3 = jnp.maximum(
            jnp.dot(p, w3_ref[...], preferred_element_type=_f32, precision=_HI)
            + b3_ref[...][None, :], 0.0)
        agg = jnp.dot(h3, wa_ref[...], preferred_element_type=_f32,
                      precision=_HI) + ba_ref[...][None, :]
        hv = jnp.dot(vec_ref[...], wv_ref[...], preferred_element_type=_f32,
                     precision=_HI) + bv_ref[...][None, :]
        z = jnp.concatenate([agg, hv], axis=1)
        z = _ln_silu(z, g1_ref[...], be1_ref[...])
        z = jnp.dot(z, wc1_ref[...], preferred_element_type=_f32,
                    precision=_HI) + bc1_ref[...][None, :]
        z = _ln_silu(z, g2_ref[...], be2_ref[...])
        z = jnp.dot(z, wc2_ref[...], preferred_element_type=_f32,
                    precision=_HI) + bc2_ref[...][None, :]
        out_ref[...] = _ln_silu(z, g3_ref[...], be3_ref[...])

    full = lambda shape: pl.BlockSpec(shape, lambda i: tuple(0 for _ in shape))
    return pl.pallas_call(
        body,
        grid=(N // BLK,),
        in_specs=[
            pl.BlockSpec((NC, BLK, D2), lambda i: (0, i, 0)),
            pl.BlockSpec((NC, BLK, D2), lambda i: (0, i, 0)),
            pl.BlockSpec((BLK, D1), lambda i: (i, 0)),
            pl.BlockSpec((BLK, 8), lambda i: (i, 0)),
            full((8, 256)), full((256,)),
            full((256, 256)), full((256,)),
            full((256, 32)), full((32,)),
            full((288,)), full((288,)),
            full((288, 512)), full((512,)),
            full((512,)), full((512,)),
            full((512, 512)), full((512,)),
            full((512,)), full((512,)),
        ],
        out_specs=pl.BlockSpec((BLK, 512), lambda i: (i, 0)),
        out_shape=jax.ShapeDtypeStruct((N, 512), _f32),
    )(m3, hs2, deg1, vecp, Wvp, bv, W3, b3, WaggT, bagg,
      g1, be1, Wc1, bc1, g2, be2, Wc2, bc2, g3, be3)


def kernel(x, edge_index, batch_index, vector, W1, b1, W2, b2, W3, b3,
           Wagg, bagg, Wv, bv, g1, be1, Wc1, bc1, g2, be2, Wc2, bc2, g3, be3):
    del batch_index
    src2d = edge_index[0].reshape(NCHUNKS, CHUNK)
    dst2d = edge_index[1].reshape(NCHUNKS, CHUNK)

    degAB = _sc_deg(dst2d)
    deg1, xs16 = _tc1(degAB, x)
    s1AB = _sc_narrow(xs16, src2d, dst2d)
    hs1 = _tc2(s1AB, deg1, x, W1[0], b1)
    m2 = _sc_wide(hs1, src2d, dst2d)
    hs2 = _tc3(m2, hs1, deg1, W2, b2)
    m3 = _sc_wide(hs2, src2d, dst2d)
    vecp = jnp.pad(vector, ((0, 0), (0, 2)))
    Wvp = jnp.pad(Wv, ((0, 2), (0, 0)))
    return _tc4(m3, hs2, deg1, vecp, Wvp, bv, W3, b3, Wagg[:256], bagg,
                g1, be1, Wc1, bc1, g2, be2, Wc2, bc2, g3, be3)
